# baseline (speedup 1.0000x reference)
import functools

import jax
import jax.numpy as jnp
from jax import lax
from jax.experimental import pallas as pl
from jax.experimental.pallas import tpu as pltpu
from jax.experimental.pallas import tpu_sc as plsc

B, S, IDIM, EMB, E, HID = 2, 2048, 1024, 128, 8, 1024
T = B * S

BM = 128
NBLK = T // BM + E
P = NBLK * BM

NC, NS = 2, 16
NW = NC * NS
CHUNK = T // NW
SUB = 64
NSUB = CHUNK // SUB
GSUB = 32
NGSUB = CHUNK // GSUB
GW = 128
PW = IDIM // 2 + GW

CT = 512
NCT = T // CT


def _router_kernel(x_ref, emb_ref, maskf_ref, rw_ref,
                   pos_ref, bexp_ref, xpk_ref, oh_scr, rank_scr, cnt_scr):
    i = pl.program_id(0)

    @pl.when(i == 0)
    def _():
        cnt_scr[...] = jnp.zeros_like(cnt_scr)

    logits = jnp.dot(emb_ref[...], rw_ref[:EMB],
                     preferred_element_type=jnp.float32)
    logits += jnp.dot(x_ref[...], rw_ref[EMB:],
                      preferred_element_type=jnp.float32)
    lmax = jnp.max(logits, axis=-1, keepdims=True)
    ex = jnp.exp(logits - lmax)
    denom = jnp.sum(ex, axis=-1, keepdims=True)
    gate = maskf_ref[...] / denom

    bits = lax.bitcast_convert_type(x_ref[...], jnp.int32)
    hi = bits[:, :IDIM // 2] + 0x8000
    lo = bits[:, IDIM // 2:] + 0x8000
    pack = (hi & jnp.int32(-65536)) | (jnp.right_shift(lo, 16) & 0xffff)
    xpk_ref[:, :IDIM // 2] = lax.bitcast_convert_type(pack, jnp.float32)
    xpk_ref[:, IDIM // 2:] = jnp.broadcast_to(gate, (CT, GW))

    idx = jnp.argmax(logits, axis=-1, keepdims=True).astype(jnp.int32)
    eio = lax.broadcasted_iota(jnp.int32, (CT, E), 1)
    oh = (eio == idx).astype(jnp.float32)
    oh_scr[pl.ds(i * CT, CT), :] = oh

    li = lax.broadcasted_iota(jnp.int32, (CT, CT), 0)
    lj = lax.broadcasted_iota(jnp.int32, (CT, CT), 1)
    lstrict = (lj < li).astype(jnp.float32)
    carry = cnt_scr[...]
    rank_scr[pl.ds(i * CT, CT), :] = (
        jnp.dot(lstrict, oh, preferred_element_type=jnp.float32) + carry)
    cnt_scr[...] = carry + jnp.sum(oh, axis=0, keepdims=True)

    @pl.when(i == NCT - 1)
    def _():
        counts = cnt_scr[...]
        nb = jnp.floor((counts + (BM - 1)) * (1.0 / BM))
        ei = lax.broadcasted_iota(jnp.int32, (E, E), 0)
        ej = lax.broadcasted_iota(jnp.int32, (E, E), 1)
        uppr = (ei < ej).astype(jnp.float32)
        cum_nb = jnp.dot(nb, uppr, preferred_element_type=jnp.float32)
        off = cum_nb * float(BM)

        pos_f = jnp.sum(oh_scr[...] * (rank_scr[...] + off),
                        axis=-1, keepdims=True)
        pos_ref[...] = pos_f.astype(jnp.int32)

        ident = (ei == ej).astype(jnp.float32)
        lstr8 = (ej < ei).astype(jnp.float32)
        nb_col = lax.dot_general(ident, nb, (((1,), (1,)), ((), ())),
                                 preferred_element_type=jnp.float32)
        cum_col = jnp.dot(lstr8, nb_col, preferred_element_type=jnp.float32)
        pio = lax.broadcasted_iota(jnp.int32, (E, NBLK), 1).astype(jnp.float32)
        ge = (pio >= cum_col).astype(jnp.float32)
        bexp_f = jnp.sum(ge, axis=0, keepdims=True) - 1.0
        total = jnp.sum(nb)
        bio = lax.broadcasted_iota(jnp.int32, (1, NBLK), 1).astype(jnp.float32)
        bexp_m = jnp.where(bio < total, bexp_f, -1.0)
        bexp_ref[...] = jnp.concatenate([bexp_f, bexp_m],
                                        axis=0).astype(jnp.int32)


def _router_call(x, emb, maskf, rw):
    return pl.pallas_call(
        _router_kernel,
        grid=(NCT,),
        in_specs=[
            pl.BlockSpec((CT, IDIM), lambda i: (i, 0)),
            pl.BlockSpec((CT, EMB), lambda i: (i, 0)),
            pl.BlockSpec((CT, 1), lambda i: (i, 0)),
            pl.BlockSpec((IDIM + EMB, E), lambda i: (0, 0)),
        ],
        out_specs=[
            pl.BlockSpec((T, 1), lambda i: (0, 0)),
            pl.BlockSpec((2, NBLK), lambda i: (0, 0)),
            pl.BlockSpec((CT, PW), lambda i: (i, 0)),
        ],
        out_shape=[
            jax.ShapeDtypeStruct((T, 1), jnp.int32),
            jax.ShapeDtypeStruct((2, NBLK), jnp.int32),
            jax.ShapeDtypeStruct((T, PW), jnp.float32),
        ],
        scratch_shapes=[pltpu.VMEM((T, E), jnp.float32),
                        pltpu.VMEM((T, E), jnp.float32),
                        pltpu.VMEM((1, E), jnp.float32)],
        compiler_params=pltpu.CompilerParams(
            dimension_semantics=("arbitrary",)),
    )(x, emb, maskf, rw)


def _sc_scatter_call(xpk, pos):
    mesh = plsc.VectorSubcoreMesh(core_axis_name="c", subcore_axis_name="s")

    @functools.partial(
        pl.kernel, mesh=mesh,
        out_type=jax.ShapeDtypeStruct((P, PW), jnp.float32),
        scratch_types=[pltpu.VMEM((NSUB, SUB), jnp.int32),
                       pltpu.VMEM((2, SUB, PW), jnp.float32),
                       pltpu.SemaphoreType.DMA,
                       pltpu.SemaphoreType.DMA,
                       pltpu.SemaphoreType.DMA],
    )
    def k(x_hbm, pos_hbm, xp_hbm, idx_v, rows_v, semld0, semld1, semx):
        wid = lax.axis_index("s") * NC + lax.axis_index("c")
        base = wid * CHUNK
        for j in range(NSUB):
            pltpu.sync_copy(pos_hbm.at[pl.ds(base + j * SUB, SUB)],
                            idx_v.at[j])
        semld = [semld0, semld1]
        cpl = [None] * NSUB
        cpx = [None] * NSUB
        for j in range(min(2, NSUB)):
            cpl[j] = pltpu.async_copy(
                x_hbm.at[pl.ds(base + j * SUB, SUB)], rows_v.at[j],
                semld[j])
        for j in range(NSUB):
            b = j % 2
            cpl[j].wait()
            cpx[j] = pltpu.async_copy(rows_v.at[b], xp_hbm.at[idx_v.at[j]],
                                      semx)
            if j + 2 < NSUB:
                cpx[j].wait()
                cpl[j + 2] = pltpu.async_copy(
                    x_hbm.at[pl.ds(base + (j + 2) * SUB, SUB)],
                    rows_v.at[b], semld[b])
        for j in range(max(NSUB - 2, 0), NSUB):
            cpx[j].wait()

    return k(xpk, pos)


def _ffn_kernel(bexp_sref, xp_ref, w1_ref, b1_ref, w2_ref, b2_ref, out_ref):
    p = pl.program_id(0)

    @pl.when(bexp_sref[1, p] >= 0)
    def _():
        v = lax.bitcast_convert_type(xp_ref[:, :IDIM // 2], jnp.int32)
        xhi = lax.bitcast_convert_type(v, jnp.float32).astype(jnp.bfloat16)
        xlo = lax.bitcast_convert_type(
            jnp.left_shift(v, 16), jnp.float32).astype(jnp.bfloat16)
        w1b = w1_ref[0].astype(jnp.bfloat16)
        h = lax.dot_general(xhi, w1b[:, :IDIM // 2],
                            (((1,), (1,)), ((), ())),
                            preferred_element_type=jnp.float32)
        h += lax.dot_general(xlo, w1b[:, IDIM // 2:],
                             (((1,), (1,)), ((), ())),
                             preferred_element_type=jnp.float32)
        h = jnp.maximum(h + b1_ref[0], 0.0).astype(jnp.bfloat16)
        w2b = w2_ref[0].astype(jnp.bfloat16)
        y = lax.dot_general(h, w2b, (((1,), (1,)), ((), ())),
                            preferred_element_type=jnp.float32)
        out_ref[...] = (y + b2_ref[0]) * xp_ref[:, IDIM // 2:IDIM // 2 + 1]


def _ffn_call(bexp, xp, w1, b1r, w2, b2r):
    grid_spec = pltpu.PrefetchScalarGridSpec(
        num_scalar_prefetch=1,
        grid=(NBLK,),
        in_specs=[
            pl.BlockSpec((BM, PW), lambda p, be: (p, 0)),
            pl.BlockSpec((1, HID, IDIM), lambda p, be: (be[0, p], 0, 0)),
            pl.BlockSpec((1, 1, HID), lambda p, be: (be[0, p], 0, 0)),
            pl.BlockSpec((1, IDIM, HID), lambda p, be: (be[0, p], 0, 0)),
            pl.BlockSpec((1, 1, IDIM), lambda p, be: (be[0, p], 0, 0)),
        ],
        out_specs=pl.BlockSpec((BM, IDIM), lambda p, be: (p, 0)),
    )
    return pl.pallas_call(
        _ffn_kernel,
        grid_spec=grid_spec,
        out_shape=jax.ShapeDtypeStruct((P, IDIM), jnp.float32),
        compiler_params=pltpu.CompilerParams(
            dimension_semantics=("parallel",)),
    )(bexp, xp, w1, b1r, w2, b2r)


def _sc_gather_call(yp, pos):
    mesh = plsc.VectorSubcoreMesh(core_axis_name="c", subcore_axis_name="s")

    @functools.partial(
        pl.kernel, mesh=mesh,
        out_type=jax.ShapeDtypeStruct((T, IDIM), jnp.float32),
        scratch_types=[pltpu.VMEM((CHUNK,), jnp.int32),
                       pltpu.VMEM((2, GSUB, IDIM), jnp.float32),
                       pltpu.SemaphoreType.DMA,
                       pltpu.SemaphoreType.DMA],
    )
    def k(yp_hbm, pos_hbm, out_hbm, idx_v, rows_v, semg, semst):
        wid = lax.axis_index("s") * NC + lax.axis_index("c")
        base = wid * CHUNK
        pltpu.sync_copy(pos_hbm.at[pl.ds(base, CHUNK)], idx_v)
        cpg = [None] * NGSUB
        cst = [None] * NGSUB
        cpg[0] = pltpu.async_copy(yp_hbm.at[idx_v.at[pl.ds(0, GSUB)]],
                                  rows_v.at[0], semg)
        for j in range(NGSUB):
            b = j % 2
            cpg[j].wait()
            cst[j] = pltpu.async_copy(
                rows_v.at[b], out_hbm.at[pl.ds(base + j * GSUB, GSUB)], semst)
            if j + 1 < NGSUB:
                if j >= 1:
                    cst[j - 1].wait()
                cpg[j + 1] = pltpu.async_copy(
                    yp_hbm.at[idx_v.at[pl.ds((j + 1) * GSUB, GSUB)]],
                    rows_v.at[1 - b], semg)
        for j in range(max(NGSUB - 2, 0), NGSUB):
            cst[j].wait()

    return k(yp, pos)


def kernel(inputs, embed, mask, router_weights, w1, b1, w2, b2):
    x = inputs.reshape(T, IDIM)
    emb = embed.reshape(T, EMB)
    maskf = mask.reshape(T, 1).astype(jnp.float32)
    b1r = b1.reshape(E, 1, HID)
    b2r = b2.reshape(E, 1, IDIM)

    pos2, bexp, xpk = _router_call(x, emb, maskf, router_weights)
    pos = pos2.reshape(T)

    xp = _sc_scatter_call(xpk, pos)
    yp = _ffn_call(bexp, xp, w1, b1r, w2, b2r)
    out = _sc_gather_call(yp, pos)
    return out.reshape(B, S, IDIM)


# speedup vs baseline: 1.2283x; 1.2283x over previous
import functools

import jax
import jax.numpy as jnp
from jax import lax
from jax.experimental import pallas as pl
from jax.experimental.pallas import tpu as pltpu
from jax.experimental.pallas import tpu_sc as plsc

B, S, IDIM, EMB, E, HID = 2, 2048, 1024, 128, 8, 1024
T = B * S

BM = 256
NBLK = T // BM + E
P = NBLK * BM

NC, NS = 2, 16
NW = NC * NS
CHUNK = T // NW
SUB = 64
NSUB = CHUNK // SUB
GSUB = 32
NGSUB = CHUNK // GSUB
GW = 128
PW = IDIM // 2 + GW

CT = 1024
NCT = T // CT


def _router_kernel(x_ref, emb_ref, maskf_ref, rw_ref,
                   pos_ref, bexp_ref, xpk_ref, oh_scr, rank_scr, cnt_scr):
    i = pl.program_id(0)

    @pl.when(i == 0)
    def _():
        cnt_scr[...] = jnp.zeros_like(cnt_scr)

    logits = jnp.dot(emb_ref[...], rw_ref[:EMB],
                     preferred_element_type=jnp.float32)
    logits += jnp.dot(x_ref[...], rw_ref[EMB:],
                      preferred_element_type=jnp.float32)
    lmax = jnp.max(logits, axis=-1, keepdims=True)
    ex = jnp.exp(logits - lmax)
    denom = jnp.sum(ex, axis=-1, keepdims=True)
    gate = maskf_ref[...] / denom

    bits = lax.bitcast_convert_type(x_ref[...], jnp.int32)
    hi = bits[:, :IDIM // 2] + 0x8000
    lo = bits[:, IDIM // 2:] + 0x8000
    pack = (hi & jnp.int32(-65536)) | (jnp.right_shift(lo, 16) & 0xffff)
    xpk_ref[:, :IDIM // 2] = lax.bitcast_convert_type(pack, jnp.float32)
    xpk_ref[:, IDIM // 2:] = jnp.broadcast_to(gate, (CT, GW))

    idx = jnp.argmax(logits, axis=-1, keepdims=True).astype(jnp.int32)
    eio = lax.broadcasted_iota(jnp.int32, (CT, E), 1)
    oh = (eio == idx).astype(jnp.float32)
    oh_scr[pl.ds(i * CT, CT), :] = oh

    li = lax.broadcasted_iota(jnp.int32, (CT, CT), 0)
    lj = lax.broadcasted_iota(jnp.int32, (CT, CT), 1)
    lstrict = (lj < li).astype(jnp.float32)
    carry = cnt_scr[...]
    rank_scr[pl.ds(i * CT, CT), :] = (
        jnp.dot(lstrict, oh, preferred_element_type=jnp.float32) + carry)
    cnt_scr[...] = carry + jnp.sum(oh, axis=0, keepdims=True)

    @pl.when(i == NCT - 1)
    def _():
        counts = cnt_scr[...]
        nb = jnp.floor((counts + (BM - 1)) * (1.0 / BM))
        ei = lax.broadcasted_iota(jnp.int32, (E, E), 0)
        ej = lax.broadcasted_iota(jnp.int32, (E, E), 1)
        uppr = (ei < ej).astype(jnp.float32)
        cum_nb = jnp.dot(nb, uppr, preferred_element_type=jnp.float32)
        off = cum_nb * float(BM)

        pos_f = jnp.sum(oh_scr[...] * (rank_scr[...] + off),
                        axis=-1, keepdims=True)
        pos_ref[...] = pos_f.astype(jnp.int32)

        ident = (ei == ej).astype(jnp.float32)
        lstr8 = (ej < ei).astype(jnp.float32)
        nb_col = lax.dot_general(ident, nb, (((1,), (1,)), ((), ())),
                                 preferred_element_type=jnp.float32)
        cum_col = jnp.dot(lstr8, nb_col, preferred_element_type=jnp.float32)
        pio = lax.broadcasted_iota(jnp.int32, (E, NBLK), 1).astype(jnp.float32)
        ge = (pio >= cum_col).astype(jnp.float32)
        bexp_f = jnp.sum(ge, axis=0, keepdims=True) - 1.0
        total = jnp.sum(nb)
        bio = lax.broadcasted_iota(jnp.int32, (1, NBLK), 1).astype(jnp.float32)
        bexp_m = jnp.where(bio < total, bexp_f, -1.0)
        bexp_ref[...] = jnp.concatenate([bexp_f, bexp_m],
                                        axis=0).astype(jnp.int32)


def _router_call(x, emb, maskf, rw):
    return pl.pallas_call(
        _router_kernel,
        grid=(NCT,),
        in_specs=[
            pl.BlockSpec((CT, IDIM), lambda i: (i, 0)),
            pl.BlockSpec((CT, EMB), lambda i: (i, 0)),
            pl.BlockSpec((CT, 1), lambda i: (i, 0)),
            pl.BlockSpec((IDIM + EMB, E), lambda i: (0, 0)),
        ],
        out_specs=[
            pl.BlockSpec((T, 1), lambda i: (0, 0)),
            pl.BlockSpec((2, NBLK), lambda i: (0, 0)),
            pl.BlockSpec((CT, PW), lambda i: (i, 0)),
        ],
        out_shape=[
            jax.ShapeDtypeStruct((T, 1), jnp.int32),
            jax.ShapeDtypeStruct((2, NBLK), jnp.int32),
            jax.ShapeDtypeStruct((T, PW), jnp.float32),
        ],
        scratch_shapes=[pltpu.VMEM((T, E), jnp.float32),
                        pltpu.VMEM((T, E), jnp.float32),
                        pltpu.VMEM((1, E), jnp.float32)],
        compiler_params=pltpu.CompilerParams(
            dimension_semantics=("arbitrary",)),
    )(x, emb, maskf, rw)


def _sc_scatter_call(xpk, pos):
    mesh = plsc.VectorSubcoreMesh(core_axis_name="c", subcore_axis_name="s")

    @functools.partial(
        pl.kernel, mesh=mesh,
        out_type=jax.ShapeDtypeStruct((P, PW), jnp.float32),
        scratch_types=[pltpu.VMEM((NSUB, SUB), jnp.int32),
                       pltpu.VMEM((2, SUB, PW), jnp.float32),
                       pltpu.SemaphoreType.DMA,
                       pltpu.SemaphoreType.DMA,
                       pltpu.SemaphoreType.DMA],
    )
    def k(x_hbm, pos_hbm, xp_hbm, idx_v, rows_v, semld0, semld1, semx):
        wid = lax.axis_index("s") * NC + lax.axis_index("c")
        base = wid * CHUNK
        for j in range(NSUB):
            pltpu.sync_copy(pos_hbm.at[pl.ds(base + j * SUB, SUB)],
                            idx_v.at[j])
        semld = [semld0, semld1]
        cpl = [None] * NSUB
        cpx = [None] * NSUB
        for j in range(min(2, NSUB)):
            cpl[j] = pltpu.async_copy(
                x_hbm.at[pl.ds(base + j * SUB, SUB)], rows_v.at[j],
                semld[j])
        for j in range(NSUB):
            b = j % 2
            cpl[j].wait()
            cpx[j] = pltpu.async_copy(rows_v.at[b], xp_hbm.at[idx_v.at[j]],
                                      semx)
            if j + 2 < NSUB:
                cpx[j].wait()
                cpl[j + 2] = pltpu.async_copy(
                    x_hbm.at[pl.ds(base + (j + 2) * SUB, SUB)],
                    rows_v.at[b], semld[b])
        for j in range(max(NSUB - 2, 0), NSUB):
            cpx[j].wait()

    return k(xpk, pos)


def _ffn_kernel(bexp_sref, xp_ref, w1_ref, b1_ref, w2_ref, b2_ref, out_ref):
    p = pl.program_id(0)

    @pl.when(bexp_sref[1, p] >= 0)
    def _():
        v = lax.bitcast_convert_type(xp_ref[:, :IDIM // 2], jnp.int32)
        xhi = lax.bitcast_convert_type(v, jnp.float32).astype(jnp.bfloat16)
        xlo = lax.bitcast_convert_type(
            jnp.left_shift(v, 16), jnp.float32).astype(jnp.bfloat16)
        w1b = w1_ref[0].astype(jnp.bfloat16)
        h = lax.dot_general(xhi, w1b[:, :IDIM // 2],
                            (((1,), (1,)), ((), ())),
                            preferred_element_type=jnp.float32)
        h += lax.dot_general(xlo, w1b[:, IDIM // 2:],
                             (((1,), (1,)), ((), ())),
                             preferred_element_type=jnp.float32)
        h = jnp.maximum(h + b1_ref[0], 0.0).astype(jnp.bfloat16)
        w2b = w2_ref[0].astype(jnp.bfloat16)
        y = lax.dot_general(h, w2b, (((1,), (1,)), ((), ())),
                            preferred_element_type=jnp.float32)
        out_ref[...] = (y + b2_ref[0]) * xp_ref[:, IDIM // 2:IDIM // 2 + 1]


def _ffn_call(bexp, xp, w1, b1r, w2, b2r):
    grid_spec = pltpu.PrefetchScalarGridSpec(
        num_scalar_prefetch=1,
        grid=(NBLK,),
        in_specs=[
            pl.BlockSpec((BM, PW), lambda p, be: (p, 0)),
            pl.BlockSpec((1, HID, IDIM), lambda p, be: (be[0, p], 0, 0)),
            pl.BlockSpec((1, 1, HID), lambda p, be: (be[0, p], 0, 0)),
            pl.BlockSpec((1, IDIM, HID), lambda p, be: (be[0, p], 0, 0)),
            pl.BlockSpec((1, 1, IDIM), lambda p, be: (be[0, p], 0, 0)),
        ],
        out_specs=pl.BlockSpec((BM, IDIM), lambda p, be: (p, 0)),
    )
    return pl.pallas_call(
        _ffn_kernel,
        grid_spec=grid_spec,
        out_shape=jax.ShapeDtypeStruct((P, IDIM), jnp.float32),
        compiler_params=pltpu.CompilerParams(
            dimension_semantics=("parallel",)),
    )(bexp, xp, w1, b1r, w2, b2r)


def _sc_gather_call(yp, pos):
    mesh = plsc.VectorSubcoreMesh(core_axis_name="c", subcore_axis_name="s")

    @functools.partial(
        pl.kernel, mesh=mesh,
        out_type=jax.ShapeDtypeStruct((T, IDIM), jnp.float32),
        scratch_types=[pltpu.VMEM((CHUNK,), jnp.int32),
                       pltpu.VMEM((2, GSUB, IDIM), jnp.float32),
                       pltpu.SemaphoreType.DMA,
                       pltpu.SemaphoreType.DMA],
    )
    def k(yp_hbm, pos_hbm, out_hbm, idx_v, rows_v, semg, semst):
        wid = lax.axis_index("s") * NC + lax.axis_index("c")
        base = wid * CHUNK
        pltpu.sync_copy(pos_hbm.at[pl.ds(base, CHUNK)], idx_v)
        cpg = [None] * NGSUB
        cst = [None] * NGSUB
        cpg[0] = pltpu.async_copy(yp_hbm.at[idx_v.at[pl.ds(0, GSUB)]],
                                  rows_v.at[0], semg)
        for j in range(NGSUB):
            b = j % 2
            cpg[j].wait()
            cst[j] = pltpu.async_copy(
                rows_v.at[b], out_hbm.at[pl.ds(base + j * GSUB, GSUB)], semst)
            if j + 1 < NGSUB:
                if j >= 1:
                    cst[j - 1].wait()
                cpg[j + 1] = pltpu.async_copy(
                    yp_hbm.at[idx_v.at[pl.ds((j + 1) * GSUB, GSUB)]],
                    rows_v.at[1 - b], semg)
        for j in range(max(NGSUB - 2, 0), NGSUB):
            cst[j].wait()

    return k(yp, pos)


def kernel(inputs, embed, mask, router_weights, w1, b1, w2, b2):
    x = inputs.reshape(T, IDIM)
    emb = embed.reshape(T, EMB)
    maskf = mask.reshape(T, 1).astype(jnp.float32)
    b1r = b1.reshape(E, 1, HID)
    b2r = b2.reshape(E, 1, IDIM)

    pos2, bexp, xpk = _router_call(x, emb, maskf, router_weights)
    pos = pos2.reshape(T)

    xp = _sc_scatter_call(xpk, pos)
    yp = _ffn_call(bexp, xp, w1, b1r, w2, b2r)
    out = _sc_gather_call(yp, pos)
    return out.reshape(B, S, IDIM)
